# baseline (speedup 1.0000x reference)
import dataclasses
import functools

import numpy as np
import jax
import jax.numpy as jnp
from jax import lax
from jax.experimental import pallas as pl
from jax.experimental.pallas import tpu as pltpu
from jax.experimental.pallas import tpu_sc as plsc

_K = 32
_L = 16


def _encode_body(x_ref, peb_ref, wt_ref, b1_ref, pre_ref):
    xc = x_ref[...] - peb_ref[...]
    pre_ref[...] = (
        jnp.dot(xc, wt_ref[...], preferred_element_type=jnp.float32)
        + b1_ref[...]
    )


def _decode_body(a_ref, w_ref, b2_ref, o_ref):
    o_ref[...] = (
        jnp.dot(a_ref[...].astype(jnp.bfloat16), w_ref[...],
                preferred_element_type=jnp.float32)
        + b2_ref[...]
    )


def _sc_mask_kernel(pre_hbm, out_hbm, in_v, out_v, su_v):
    tokens, d = pre_hbm.shape
    nw = 32
    rows_per_w = tokens // nw
    chunk = in_v.shape[0]
    nvec = d // _L
    minint = jnp.int32(np.int32(-(2**31)))
    maxint = jnp.int32(np.int32(2**31 - 1))

    wid = lax.axis_index("s") * 2 + lax.axis_index("c")
    base = wid * rows_per_w

    @pl.loop(0, rows_per_w // chunk)
    def _chunk_loop(ci):
        row0 = base + ci * chunk
        pltpu.sync_copy(pre_hbm.at[pl.ds(row0, chunk), :], in_v)

        @pl.loop(0, chunk)
        def _row_loop(r):
            @pl.loop(0, nvec)
            def _su_loop(j):
                v = in_v[r, pl.ds(j * _L, _L)]
                k = plsc.bitcast(v, jnp.int32)
                su_v[pl.ds(j * _L, _L)] = k ^ (
                    (k >> jnp.int32(31)) & jnp.int32(0x7FFFFFFF)
                )

            def _count_ge(sc):
                def body(j, acc):
                    s = su_v[pl.ds(j * _L, _L)]
                    return acc + jnp.where(
                        s >= sc, jnp.int32(1), jnp.int32(0)
                    )
                acc = lax.fori_loop(0, nvec, body,
                                    jnp.zeros((_L,), jnp.int32))
                return jnp.sum(acc, axis=0)

            def cond(carry):
                i, p, done = carry
                return jnp.logical_and(i >= 0, jnp.logical_not(done))

            def step(carry):
                i, p, done = carry
                bit = lax.shift_left(jnp.int32(1), i)
                c = p | bit
                cnt = _count_ge(c ^ minint)
                p2 = jnp.where(cnt >= _K, c, p)
                return (i - 1, p2, cnt == _K)

            i0 = jnp.int32(31)
            p0 = jnp.int32(0)
            i_f, p_f, done_f = lax.while_loop(
                cond, step, (i0, p0, jnp.bool_(False)))

            sp0 = p_f ^ minint

            def masked_min(sp):
                def body(j, acc):
                    s = su_v[pl.ds(j * _L, _L)]
                    return jnp.minimum(acc, jnp.where(s >= sp, s, maxint))
                acc = lax.fori_loop(0, nvec, body,
                                    jnp.full((_L,), maxint, jnp.int32))
                return jnp.min(acc, axis=0)

            sp = jnp.where(done_f, masked_min(sp0), sp0)

            @pl.loop(0, nvec)
            def _mask_loop(j):
                v = in_v[r, pl.ds(j * _L, _L)]
                s = su_v[pl.ds(j * _L, _L)]
                out_v[r, pl.ds(j * _L, _L)] = jnp.where(
                    s >= sp, v, jnp.float32(0.0))

        pltpu.sync_copy(out_v, out_hbm.at[pl.ds(row0, chunk), :])


def _sc_mask(pre):
    tokens, d = pre.shape
    chunk = 8
    mesh = plsc.VectorSubcoreMesh(core_axis_name="c", subcore_axis_name="s")
    cp = pltpu.CompilerParams()
    if "needs_layout_passes" in pltpu.CompilerParams.__dataclass_fields__:
        cp = dataclasses.replace(cp, needs_layout_passes=False)
    f = pl.kernel(
        _sc_mask_kernel,
        out_type=jax.ShapeDtypeStruct((tokens, d), jnp.float32),
        mesh=mesh,
        compiler_params=cp,
        scratch_types=[
            pltpu.VMEM((chunk, d), jnp.float32),
            pltpu.VMEM((chunk, d), jnp.float32),
            pltpu.VMEM((d,), jnp.int32),
        ],
    )
    return f(pre)


def kernel(x, pre_encode_b, W, WT, b1, b2):
    tokens, input_size = x.shape
    hidden = WT.shape[1]
    bt = 256

    pre = pl.pallas_call(
        _encode_body,
        grid=(tokens // bt,),
        in_specs=[
            pl.BlockSpec((bt, input_size), lambda i: (i, 0)),
            pl.BlockSpec((1, hidden), lambda i: (0, 0)),
            pl.BlockSpec((input_size, hidden), lambda i: (0, 0)),
            pl.BlockSpec((1, hidden), lambda i: (0, 0)),
        ],
        out_specs=pl.BlockSpec((bt, hidden), lambda i: (i, 0)),
        out_shape=jax.ShapeDtypeStruct((tokens, hidden), jnp.float32),
        compiler_params=pltpu.CompilerParams(
            dimension_semantics=("parallel",),
        ),
    )(x, pre_encode_b.reshape(1, hidden), WT, b1.reshape(1, hidden))

    masked = _sc_mask(pre)

    out = pl.pallas_call(
        _decode_body,
        grid=(tokens // bt,),
        in_specs=[
            pl.BlockSpec((bt, hidden), lambda i: (i, 0)),
            pl.BlockSpec((hidden, input_size), lambda i: (0, 0)),
            pl.BlockSpec((1, input_size), lambda i: (0, 0)),
        ],
        out_specs=pl.BlockSpec((bt, input_size), lambda i: (i, 0)),
        out_shape=jax.ShapeDtypeStruct((tokens, input_size), jnp.float32),
        compiler_params=pltpu.CompilerParams(
            dimension_semantics=("parallel",),
        ),
    )(masked, W.astype(jnp.bfloat16), b2.reshape(1, input_size))

    return out


# speedup vs baseline: 6.1974x; 6.1974x over previous
import functools

import numpy as np
import jax
import jax.numpy as jnp
from jax.experimental import pallas as pl
from jax.experimental.pallas import tpu as pltpu

_K = 32


def _fused_body(x_ref, peb_ref, w_ref, wt_ref, b1_ref, b2_ref, o_ref, *, k):
    xc = x_ref[...] - peb_ref[...]
    pre = jnp.dot(xc, wt_ref[...], preferred_element_type=jnp.float32)
    pre = pre + b1_ref[...]

    b = jax.lax.bitcast_convert_type(pre, jnp.int32)
    su = b ^ ((b >> jnp.int32(31)) & jnp.int32(0x7FFFFFFF))
    minint = jnp.int32(np.int32(-2**31))

    rows = pre.shape[0]
    p = jnp.zeros((rows, 1), jnp.int32)
    for i in range(31, -1, -1):
        bit = jnp.int32(np.uint32(1 << i).astype(np.int32))
        c = p | bit
        sc = c ^ minint
        cnt = jnp.sum((su >= sc).astype(jnp.int32), axis=1, keepdims=True)
        p = jnp.where(cnt >= k, c, p)
    sp = p ^ minint
    masked = jnp.where(su >= sp, pre, jnp.float32(0.0))

    out = jnp.dot(masked, w_ref[...], preferred_element_type=jnp.float32)
    o_ref[...] = out + b2_ref[...]


def kernel(x, pre_encode_b, W, WT, b1, b2):
    tokens, input_size = x.shape
    hidden = WT.shape[1]
    bt = 256
    grid = (tokens // bt,)
    out = pl.pallas_call(
        functools.partial(_fused_body, k=_K),
        grid=grid,
        in_specs=[
            pl.BlockSpec((bt, input_size), lambda i: (i, 0)),
            pl.BlockSpec((1, hidden), lambda i: (0, 0)),
            pl.BlockSpec((hidden, input_size), lambda i: (0, 0)),
            pl.BlockSpec((input_size, hidden), lambda i: (0, 0)),
            pl.BlockSpec((1, hidden), lambda i: (0, 0)),
            pl.BlockSpec((1, input_size), lambda i: (0, 0)),
        ],
        out_specs=pl.BlockSpec((bt, input_size), lambda i: (i, 0)),
        out_shape=jax.ShapeDtypeStruct((tokens, input_size), jnp.float32),
        compiler_params=pltpu.CompilerParams(
            dimension_semantics=("parallel",),
        ),
    )(
        x,
        pre_encode_b.reshape(1, hidden),
        W,
        WT,
        b1.reshape(1, hidden),
        b2.reshape(1, input_size),
    )
    return out
